# Initial kernel scaffold; baseline (speedup 1.0000x reference)
import jax
import jax.numpy as jnp
from jax.experimental import pallas as pl


def kernel(x, init_w, init_b, b0_c0w, b0_c0b, b0_fw, b0_fb, b0_gw, b0_gb, b0_rw, b0_rb, b1_c0w, b1_c0b, b1_fw, b1_fb, b1_gw, b1_gb, b1_rw, b1_rb, b2_c0w, b2_c0b, b2_fw, b2_fb, b2_gw, b2_gb, b2_rw, b2_rb, b3_c0w, b3_c0b, b3_fw, b3_fb, b3_gw, b3_gb, b3_rw, b3_rb):
    raise NotImplementedError("write your pallas kernel here")


# speedup vs baseline: 1.4433x; 1.4433x over previous
import functools

import jax
import jax.numpy as jnp
import numpy as np
from jax.experimental import pallas as pl
from jax.experimental.pallas import tpu as pltpu


def _shift(x, offset):
    if offset == 0:
        return x
    T = x.shape[-1]
    lane = jax.lax.broadcasted_iota(jnp.int32, x.shape, x.ndim - 1)
    if offset > 0:
        rolled = pltpu.roll(x, shift=T - offset, axis=x.ndim - 1)
        return jnp.where(lane < T - offset, rolled, 0)
    s = -offset
    rolled = pltpu.roll(x, shift=s, axis=x.ndim - 1)
    return jnp.where(lane >= s, rolled, 0)


def _enc_kernel(x_ref, iw_ref, ib_ref, c0w_ref, c0b_ref, w_ref, fgb_ref,
                rw_ref, rb_ref, p_ref, o_ref, *, num_layers):
    xin = x_ref[0]
    h = iw_ref[1] * xin
    h = h + iw_ref[0] * _shift(xin, -1)
    h = h + iw_ref[2] * _shift(xin, 1)
    h = h + ib_ref[...]
    hb = h.astype(jnp.bfloat16)
    x = jnp.dot(c0w_ref[0], hb, preferred_element_type=jnp.float32)
    x = x + c0b_ref[0]
    res = x
    F = x.shape[0]
    for j in range(num_layers):
        d = 1 << j
        xb = x.astype(jnp.bfloat16)
        sm = _shift(xb, -d)
        sp = _shift(xb, d)
        cat = jnp.concatenate([sm, xb, sp], axis=0)
        y = jnp.dot(w_ref[0, j], cat, preferred_element_type=jnp.float32)
        y = y + fgb_ref[0, j]
        f = y[:F]
        g = y[F:]
        z = jnp.tanh(f) * (1.0 / (1.0 + jnp.exp(-g)))
        xn = jnp.dot(rw_ref[0, j], z.astype(jnp.bfloat16),
                     preferred_element_type=jnp.float32)
        xn = xn + rb_ref[0, j]
        res = res + xn
        x = xn
    pooled = jnp.dot(res.astype(jnp.bfloat16), p_ref[...],
                     preferred_element_type=jnp.float32)
    o_ref[0, 0] = pooled


def _pool_matrix(T, S):
    P = np.zeros((T, S), np.float32)
    for i in range(S):
        start = (i * T) // S
        end = -((-(i + 1) * T) // S)
        P[start:end, i] = 1.0 / (end - start)
    return P


def _stack_gate_weights(fw, gw, L, F):
    fwr = fw.reshape(L, 3, F, F)
    gwr = gw.reshape(L, 3, F, F)
    w = jnp.concatenate([fwr, gwr], axis=2)
    return w.transpose(0, 2, 1, 3).reshape(L, 2 * F, 3 * F)


def _forward(x, init_w, init_b, blocks, out_size):
    B, Cin, T = x.shape
    N = B * Cin
    K, F, _ = init_w.shape
    nst = len(blocks)
    L = blocks[0]["fw"].shape[0] // K
    bf16 = jnp.bfloat16

    c0w = jnp.stack([b["c0w"] for b in blocks]).astype(bf16)
    c0b = jnp.stack([b["c0b"] for b in blocks])
    W = jnp.stack([_stack_gate_weights(b["fw"], b["gw"], L, F)
                   for b in blocks]).astype(bf16)
    fgb = jnp.stack([jnp.concatenate([b["fb"], b["gb"]], axis=1)
                     for b in blocks])
    rw = jnp.stack([b["rw"] for b in blocks]).astype(bf16)
    rb = jnp.stack([b["rb"] for b in blocks])
    P = jnp.asarray(_pool_matrix(T, out_size)).astype(bf16)
    xr = x.reshape(N, 1, T).astype(jnp.float32)

    kern = functools.partial(_enc_kernel, num_layers=L)
    out = pl.pallas_call(
        kern,
        grid=(nst, N),
        in_specs=[
            pl.BlockSpec((1, 1, T), lambda s, r: (r, 0, 0)),
            pl.BlockSpec((K, F, 1), lambda s, r: (0, 0, 0)),
            pl.BlockSpec((F, 1), lambda s, r: (0, 0)),
            pl.BlockSpec((1, F, F), lambda s, r: (s, 0, 0)),
            pl.BlockSpec((1, F, 1), lambda s, r: (s, 0, 0)),
            pl.BlockSpec((1, L, 2 * F, 3 * F), lambda s, r: (s, 0, 0, 0)),
            pl.BlockSpec((1, L, 2 * F, 1), lambda s, r: (s, 0, 0, 0)),
            pl.BlockSpec((1, L, F, F), lambda s, r: (s, 0, 0, 0)),
            pl.BlockSpec((1, L, F, 1), lambda s, r: (s, 0, 0, 0)),
            pl.BlockSpec((T, out_size), lambda s, r: (0, 0)),
        ],
        out_specs=pl.BlockSpec((1, 1, F, out_size),
                               lambda s, r: (r // Cin, s, r % Cin, 0)),
        out_shape=jax.ShapeDtypeStruct((B, nst, Cin * F, out_size),
                                       jnp.float32),
        compiler_params=pltpu.CompilerParams(
            dimension_semantics=("parallel", "parallel")),
    )(xr, init_w, init_b, c0w, c0b, W, fgb, rw, rb, P)
    return out


def kernel(x, init_w, init_b,
           b0_c0w, b0_c0b, b0_fw, b0_fb, b0_gw, b0_gb, b0_rw, b0_rb,
           b1_c0w, b1_c0b, b1_fw, b1_fb, b1_gw, b1_gb, b1_rw, b1_rb,
           b2_c0w, b2_c0b, b2_fw, b2_fb, b2_gw, b2_gb, b2_rw, b2_rb,
           b3_c0w, b3_c0b, b3_fw, b3_fb, b3_gw, b3_gb, b3_rw, b3_rb):
    blocks = [
        {"c0w": b0_c0w, "c0b": b0_c0b, "fw": b0_fw, "fb": b0_fb,
         "gw": b0_gw, "gb": b0_gb, "rw": b0_rw, "rb": b0_rb},
        {"c0w": b1_c0w, "c0b": b1_c0b, "fw": b1_fw, "fb": b1_fb,
         "gw": b1_gw, "gb": b1_gb, "rw": b1_rw, "rb": b1_rb},
        {"c0w": b2_c0w, "c0b": b2_c0b, "fw": b2_fw, "fb": b2_fb,
         "gw": b2_gw, "gb": b2_gb, "rw": b2_rw, "rb": b2_rb},
        {"c0w": b3_c0w, "c0b": b3_c0b, "fw": b3_fw, "fb": b3_fb,
         "gw": b3_gw, "gb": b3_gb, "rw": b3_rw, "rb": b3_rb},
    ]
    return _forward(x, init_w, init_b, blocks, 320)


# speedup vs baseline: 1.5014x; 1.0402x over previous
import functools

import jax
import jax.numpy as jnp
import numpy as np
from jax.experimental import pallas as pl
from jax.experimental.pallas import tpu as pltpu


def _shift(x, offset):
    if offset == 0:
        return x
    T = x.shape[-1]
    lane = jax.lax.broadcasted_iota(jnp.int32, x.shape, x.ndim - 1)
    if offset > 0:
        rolled = pltpu.roll(x, shift=T - offset, axis=x.ndim - 1)
        return jnp.where(lane < T - offset, rolled, 0)
    s = -offset
    rolled = pltpu.roll(x, shift=s, axis=x.ndim - 1)
    return jnp.where(lane >= s, rolled, 0)


def _row_chain(xin, iw_ref, c0w_ref, w_ref, rw_ref, p_ref, ones8, num_layers):
    h = iw_ref[1] * xin
    h = h + iw_ref[0] * _shift(xin, -1)
    h = h + iw_ref[2] * _shift(xin, 1)
    hcat = jnp.concatenate([h.astype(jnp.bfloat16), ones8], axis=0)
    x = jnp.dot(c0w_ref[0], hcat, preferred_element_type=jnp.float32)
    res = x
    F = x.shape[0]
    for j in range(num_layers):
        d = 1 << j
        xb = x.astype(jnp.bfloat16)
        sm = _shift(xb, -d)
        sp = _shift(xb, d)
        cat = jnp.concatenate([sm, xb, sp, ones8], axis=0)
        y = jnp.dot(w_ref[0, j], cat, preferred_element_type=jnp.float32)
        z = jnp.tanh(y[:F]) * (1.0 / (1.0 + jnp.exp(-y[F:])))
        zcat = jnp.concatenate([z.astype(jnp.bfloat16), ones8], axis=0)
        xn = jnp.dot(rw_ref[0, j], zcat, preferred_element_type=jnp.float32)
        res = res + xn
        x = xn
    return jnp.dot(res.astype(jnp.bfloat16), p_ref[...],
                   preferred_element_type=jnp.float32)


def _enc_kernel(x_ref, iw_ref, c0w_ref, w_ref, rw_ref, p_ref, o_ref, *,
                num_layers, rows):
    T = x_ref.shape[-1]
    sub = jax.lax.broadcasted_iota(jnp.int32, (8, T), 0)
    ones8 = jnp.where(sub == 0, 1, 0).astype(jnp.bfloat16)
    pooled = [_row_chain(x_ref[i], iw_ref, c0w_ref, w_ref, rw_ref, p_ref,
                         ones8, num_layers) for i in range(rows)]
    o_ref[0, 0] = jnp.concatenate(pooled, axis=0)


def _pool_matrix(T, S):
    P = np.zeros((T, S), np.float32)
    for i in range(S):
        start = (i * T) // S
        end = -((-(i + 1) * T) // S)
        P[start:end, i] = 1.0 / (end - start)
    return P


def _with_bias_cols(w, b):
    pad = [(0, 0)] * (w.ndim - 1) + [(0, 7)]
    return jnp.concatenate([w, jnp.pad(b, pad)], axis=-1)


def _stack_gate_weights(fw, gw, L, F):
    fwr = fw.reshape(L, 3, F, F)
    gwr = gw.reshape(L, 3, F, F)
    w = jnp.concatenate([fwr, gwr], axis=2)
    return w.transpose(0, 2, 1, 3).reshape(L, 2 * F, 3 * F)


def _forward(x, init_w, init_b, blocks, out_size):
    B, Cin, T = x.shape
    N = B * Cin
    K, F, _ = init_w.shape
    nst = len(blocks)
    L = blocks[0]["fw"].shape[0] // K
    bf16 = jnp.bfloat16
    rows = 2
    assert Cin == rows and N % rows == 0

    c0w = jnp.stack([
        _with_bias_cols(b["c0w"], b["c0b"] + b["c0w"] @ init_b)
        for b in blocks]).astype(bf16)
    W = jnp.stack([
        _with_bias_cols(_stack_gate_weights(b["fw"], b["gw"], L, F),
                        jnp.concatenate([b["fb"], b["gb"]], axis=1))
        for b in blocks]).astype(bf16)
    rw = jnp.stack([_with_bias_cols(b["rw"], b["rb"])
                    for b in blocks]).astype(bf16)
    P = jnp.asarray(_pool_matrix(T, out_size)).astype(bf16)
    xr = x.reshape(N, 1, T).astype(jnp.float32)

    kern = functools.partial(_enc_kernel, num_layers=L, rows=rows)
    out = pl.pallas_call(
        kern,
        grid=(nst, N // rows),
        in_specs=[
            pl.BlockSpec((rows, 1, T), lambda s, q: (q, 0, 0)),
            pl.BlockSpec((K, F, 1), lambda s, q: (0, 0, 0)),
            pl.BlockSpec((1, F, F + 8), lambda s, q: (s, 0, 0)),
            pl.BlockSpec((1, L, 2 * F, 3 * F + 8), lambda s, q: (s, 0, 0, 0)),
            pl.BlockSpec((1, L, F, F + 8), lambda s, q: (s, 0, 0, 0)),
            pl.BlockSpec((T, out_size), lambda s, q: (0, 0)),
        ],
        out_specs=pl.BlockSpec((1, 1, rows * F, out_size),
                               lambda s, q: (q, s, 0, 0)),
        out_shape=jax.ShapeDtypeStruct((B, nst, Cin * F, out_size),
                                       jnp.float32),
        compiler_params=pltpu.CompilerParams(
            dimension_semantics=("parallel", "parallel")),
    )(xr, init_w, c0w, W, rw, P)
    return out


def kernel(x, init_w, init_b,
           b0_c0w, b0_c0b, b0_fw, b0_fb, b0_gw, b0_gb, b0_rw, b0_rb,
           b1_c0w, b1_c0b, b1_fw, b1_fb, b1_gw, b1_gb, b1_rw, b1_rb,
           b2_c0w, b2_c0b, b2_fw, b2_fb, b2_gw, b2_gb, b2_rw, b2_rb,
           b3_c0w, b3_c0b, b3_fw, b3_fb, b3_gw, b3_gb, b3_rw, b3_rb):
    blocks = [
        {"c0w": b0_c0w, "c0b": b0_c0b, "fw": b0_fw, "fb": b0_fb,
         "gw": b0_gw, "gb": b0_gb, "rw": b0_rw, "rb": b0_rb},
        {"c0w": b1_c0w, "c0b": b1_c0b, "fw": b1_fw, "fb": b1_fb,
         "gw": b1_gw, "gb": b1_gb, "rw": b1_rw, "rb": b1_rb},
        {"c0w": b2_c0w, "c0b": b2_c0b, "fw": b2_fw, "fb": b2_fb,
         "gw": b2_gw, "gb": b2_gb, "rw": b2_rw, "rb": b2_rb},
        {"c0w": b3_c0w, "c0b": b3_c0b, "fw": b3_fw, "fb": b3_fb,
         "gw": b3_gw, "gb": b3_gb, "rw": b3_rw, "rb": b3_rb},
    ]
    return _forward(x, init_w, init_b, blocks, 320)


# speedup vs baseline: 1.6748x; 1.1155x over previous
import functools

import jax
import jax.numpy as jnp
import numpy as np
from jax.experimental import pallas as pl
from jax.experimental.pallas import tpu as pltpu


def _shift(x, offset):
    if offset == 0:
        return x
    T = x.shape[-1]
    lane = jax.lax.broadcasted_iota(jnp.int32, x.shape, x.ndim - 1)
    if offset > 0:
        rolled = pltpu.roll(x, shift=T - offset, axis=x.ndim - 1)
        return jnp.where(lane < T - offset, rolled, 0)
    s = -offset
    rolled = pltpu.roll(x, shift=s, axis=x.ndim - 1)
    return jnp.where(lane >= s, rolled, 0)


def _row_chain(xin, iw_ref, c0w_ref, w_ref, rwcat_ref, rbsum_ref, p_ref,
               lane_t, num_layers):
    T = xin.shape[-1]
    sub = jax.lax.broadcasted_iota(jnp.int32, (8, T), 0)
    h = iw_ref[1] * xin
    h = h + iw_ref[0] * _shift(xin, -1)
    h = h + iw_ref[2] * _shift(xin, 1)
    ones8 = jnp.where(sub == 0, 1, 0).astype(jnp.bfloat16)
    hcat = jnp.concatenate([h.astype(jnp.bfloat16), ones8], axis=0)
    x0 = jnp.dot(c0w_ref[0], hcat, preferred_element_type=jnp.float32)
    F = x0.shape[0]
    prev = x0.astype(jnp.bfloat16)
    zs = []
    for j in range(num_layers):
        d = 1 << j
        ind = jnp.where(
            sub == 0, 1,
            jnp.where((sub == 1) & (lane_t < d), 1,
                      jnp.where((sub == 2) & (lane_t >= T - d), 1, 0)))
        cat = jnp.concatenate(
            [_shift(prev, -d), prev, _shift(prev, d),
             ind.astype(jnp.bfloat16)], axis=0)
        y = jnp.dot(w_ref[0, j], cat, preferred_element_type=jnp.float32)
        z = jnp.tanh(y[:F]) * (1.0 / (1.0 + jnp.exp(-y[F:])))
        prev = z.astype(jnp.bfloat16)
        zs.append(prev)
    Z = jnp.concatenate(zs, axis=0)
    radd = jnp.dot(rwcat_ref[0], Z, preferred_element_type=jnp.float32)
    res = x0 + radd
    pooled = jnp.dot(res.astype(jnp.bfloat16), p_ref[...],
                     preferred_element_type=jnp.float32)
    return pooled + rbsum_ref[0]


def _enc_kernel(x_ref, iw_ref, c0w_ref, w_ref, rwcat_ref, rbsum_ref, p_ref,
                o_ref, *, num_layers, rows):
    T = x_ref.shape[-1]
    lane_t = jax.lax.broadcasted_iota(jnp.int32, (8, T), 1)
    pooled = [_row_chain(x_ref[i], iw_ref, c0w_ref, w_ref, rwcat_ref,
                         rbsum_ref, p_ref, lane_t, num_layers)
              for i in range(rows)]
    o_ref[0, 0] = jnp.concatenate(pooled, axis=0)


def _pool_matrix(T, S):
    P = np.zeros((T, S), np.float32)
    for i in range(S):
        start = (i * T) // S
        end = -((-(i + 1) * T) // S)
        P[start:end, i] = 1.0 / (end - start)
    return P


def _stack_gate_weights(fw, gw, L, F):
    fwr = fw.reshape(L, 3, F, F)
    gwr = gw.reshape(L, 3, F, F)
    w = jnp.concatenate([fwr, gwr], axis=2)
    return w.transpose(0, 2, 1, 3)


def _prep_block(blk, init_b, L, F):
    G = _stack_gate_weights(blk["fw"], blk["gw"], L, F)
    fgb = jnp.concatenate([blk["fb"], blk["gb"]], axis=1)
    M = jnp.concatenate(
        [G[:1], jnp.einsum("jakf,jfg->jakg", G[1:], blk["rw"][:-1])],
        axis=0)
    b = jnp.einsum("jakf,jf->jak", G[1:], blk["rb"][:-1, :, 0])
    zpad = jnp.zeros((1, 2 * F), jnp.float32)
    bias = fgb[:, :, 0] + jnp.concatenate([zpad, b.sum(-1)], axis=0)
    corr_l = jnp.concatenate([zpad, -b[:, :, 0]], axis=0)
    corr_r = jnp.concatenate([zpad, -b[:, :, 2]], axis=0)
    pad = jnp.zeros((L, 2 * F, 5), jnp.float32)
    Wm = jnp.concatenate(
        [M.reshape(L, 2 * F, 3 * F), bias[..., None], corr_l[..., None],
         corr_r[..., None], pad], axis=-1)
    c0bias = blk["c0b"] + blk["c0w"] @ init_b
    c0 = jnp.concatenate(
        [blk["c0w"], c0bias, jnp.zeros((F, 7), jnp.float32)], axis=-1)
    rwcat = blk["rw"].transpose(1, 0, 2).reshape(F, L * F)
    rbsum = blk["rb"].sum(0)
    return c0, Wm, rwcat, rbsum


def _forward(x, init_w, init_b, blocks, out_size):
    B, Cin, T = x.shape
    N = B * Cin
    K, F, _ = init_w.shape
    nst = len(blocks)
    L = blocks[0]["fw"].shape[0] // K
    bf16 = jnp.bfloat16
    rows = 2
    assert Cin == rows and N % rows == 0

    prepped = [_prep_block(b, init_b, L, F) for b in blocks]
    c0w = jnp.stack([p[0] for p in prepped]).astype(bf16)
    W = jnp.stack([p[1] for p in prepped]).astype(bf16)
    rwcat = jnp.stack([p[2] for p in prepped]).astype(bf16)
    rbsum = jnp.stack([p[3] for p in prepped])
    P = jnp.asarray(_pool_matrix(T, out_size)).astype(bf16)
    xr = x.reshape(N, 1, T).astype(jnp.float32)

    kern = functools.partial(_enc_kernel, num_layers=L, rows=rows)
    out = pl.pallas_call(
        kern,
        grid=(nst, N // rows),
        in_specs=[
            pl.BlockSpec((rows, 1, T), lambda s, q: (q, 0, 0)),
            pl.BlockSpec((K, F, 1), lambda s, q: (0, 0, 0)),
            pl.BlockSpec((1, F, F + 8), lambda s, q: (s, 0, 0)),
            pl.BlockSpec((1, L, 2 * F, 3 * F + 8), lambda s, q: (s, 0, 0, 0)),
            pl.BlockSpec((1, F, L * F), lambda s, q: (s, 0, 0)),
            pl.BlockSpec((1, F, 1), lambda s, q: (s, 0, 0)),
            pl.BlockSpec((T, out_size), lambda s, q: (0, 0)),
        ],
        out_specs=pl.BlockSpec((1, 1, rows * F, out_size),
                               lambda s, q: (q, s, 0, 0)),
        out_shape=jax.ShapeDtypeStruct((B, nst, Cin * F, out_size),
                                       jnp.float32),
        compiler_params=pltpu.CompilerParams(
            dimension_semantics=("parallel", "parallel")),
    )(xr, init_w, c0w, W, rwcat, rbsum, P)
    return out


def kernel(x, init_w, init_b,
           b0_c0w, b0_c0b, b0_fw, b0_fb, b0_gw, b0_gb, b0_rw, b0_rb,
           b1_c0w, b1_c0b, b1_fw, b1_fb, b1_gw, b1_gb, b1_rw, b1_rb,
           b2_c0w, b2_c0b, b2_fw, b2_fb, b2_gw, b2_gb, b2_rw, b2_rb,
           b3_c0w, b3_c0b, b3_fw, b3_fb, b3_gw, b3_gb, b3_rw, b3_rb):
    blocks = [
        {"c0w": b0_c0w, "c0b": b0_c0b, "fw": b0_fw, "fb": b0_fb,
         "gw": b0_gw, "gb": b0_gb, "rw": b0_rw, "rb": b0_rb},
        {"c0w": b1_c0w, "c0b": b1_c0b, "fw": b1_fw, "fb": b1_fb,
         "gw": b1_gw, "gb": b1_gb, "rw": b1_rw, "rb": b1_rb},
        {"c0w": b2_c0w, "c0b": b2_c0b, "fw": b2_fw, "fb": b2_fb,
         "gw": b2_gw, "gb": b2_gb, "rw": b2_rw, "rb": b2_rb},
        {"c0w": b3_c0w, "c0b": b3_c0b, "fw": b3_fw, "fb": b3_fb,
         "gw": b3_gw, "gb": b3_gb, "rw": b3_rw, "rb": b3_rb},
    ]
    return _forward(x, init_w, init_b, blocks, 320)


# speedup vs baseline: 1.7059x; 1.0185x over previous
import functools

import jax
import jax.numpy as jnp
import numpy as np
from jax.experimental import pallas as pl
from jax.experimental.pallas import tpu as pltpu


def _shift(x, offset):
    if offset == 0:
        return x
    T = x.shape[-1]
    lane = jax.lax.broadcasted_iota(jnp.int32, x.shape, x.ndim - 1)
    if offset > 0:
        rolled = pltpu.roll(x, shift=T - offset, axis=x.ndim - 1)
        return jnp.where(lane < T - offset, rolled, 0)
    s = -offset
    rolled = pltpu.roll(x, shift=s, axis=x.ndim - 1)
    return jnp.where(lane >= s, rolled, 0)


def _row_chain(xin, iw_ref, c0w_ref, w_ref, rwcat_ref, rbsum_ref, p_ref,
               lane_t, num_layers):
    T = xin.shape[-1]
    sub = jax.lax.broadcasted_iota(jnp.int32, (8, T), 0)
    h = iw_ref[1] * xin
    h = h + iw_ref[0] * _shift(xin, -1)
    h = h + iw_ref[2] * _shift(xin, 1)
    ones8 = jnp.where(sub == 0, 1, 0).astype(jnp.bfloat16)
    hcat = jnp.concatenate([h.astype(jnp.bfloat16), ones8], axis=0)
    x0 = jnp.dot(c0w_ref[0], hcat, preferred_element_type=jnp.float32)
    F = x0.shape[0]
    prev = x0.astype(jnp.bfloat16)
    zs = []
    for j in range(num_layers):
        d = 1 << j
        ind = jnp.where(
            sub == 0, 1,
            jnp.where((sub == 1) & (lane_t < d), 1,
                      jnp.where((sub == 2) & (lane_t >= T - d), 1, 0)))
        cat = jnp.concatenate(
            [_shift(prev, -d), prev, _shift(prev, d),
             ind.astype(jnp.bfloat16)], axis=0)
        y = jnp.dot(w_ref[0, j], cat, preferred_element_type=jnp.float32)
        z = jnp.tanh(y[:F]) * (1.0 / (1.0 + jnp.exp2(y[F:])))
        prev = z.astype(jnp.bfloat16)
        zs.append(prev)
    Z = jnp.concatenate(zs, axis=0)
    radd = jnp.dot(rwcat_ref[0], Z, preferred_element_type=jnp.float32)
    res = x0 + radd
    pooled = jnp.dot(res.astype(jnp.bfloat16), p_ref[...],
                     preferred_element_type=jnp.float32)
    return pooled + rbsum_ref[0]


def _enc_kernel(x_ref, iw_ref, c0w_ref, w_ref, rwcat_ref, rbsum_ref, p_ref,
                o_ref, *, num_layers, rows):
    T = x_ref.shape[-1]
    lane_t = jax.lax.broadcasted_iota(jnp.int32, (8, T), 1)
    pooled = [_row_chain(x_ref[i], iw_ref, c0w_ref, w_ref, rwcat_ref,
                         rbsum_ref, p_ref, lane_t, num_layers)
              for i in range(rows)]
    o_ref[0, 0] = jnp.concatenate(pooled, axis=0)


def _pool_matrix(T, S):
    P = np.zeros((T, S), np.float32)
    for i in range(S):
        start = (i * T) // S
        end = -((-(i + 1) * T) // S)
        P[start:end, i] = 1.0 / (end - start)
    return P


def _stack_gate_weights(fw, gw, L, F):
    fwr = fw.reshape(L, 3, F, F)
    gwr = gw.reshape(L, 3, F, F)
    w = jnp.concatenate([fwr, gwr], axis=2)
    return w.transpose(0, 2, 1, 3)


def _prep_block(blk, init_b, L, F):
    bf16 = jnp.bfloat16
    G = _stack_gate_weights(blk["fw"].astype(bf16), blk["gw"].astype(bf16),
                            L, F)
    fgb = jnp.concatenate([blk["fb"], blk["gb"]], axis=1)
    M = jnp.concatenate(
        [G[:1], jnp.einsum("jakf,jfg->jakg", G[1:],
                           blk["rw"][:-1].astype(bf16),
                           preferred_element_type=jnp.float32).astype(bf16)],
        axis=0)
    b = jnp.einsum("jakf,jf->jak", G[1:].astype(jnp.float32),
                   blk["rb"][:-1, :, 0])
    zpad = jnp.zeros((1, 2 * F), jnp.float32)
    bias = fgb[:, :, 0] + jnp.concatenate([zpad, b.sum(-1)], axis=0)
    corr_l = jnp.concatenate([zpad, -b[:, :, 0]], axis=0)
    corr_r = jnp.concatenate([zpad, -b[:, :, 2]], axis=0)
    pad = jnp.zeros((L, 2 * F, 5), bf16)
    Wm = jnp.concatenate(
        [M.reshape(L, 2 * F, 3 * F),
         bias[..., None].astype(bf16), corr_l[..., None].astype(bf16),
         corr_r[..., None].astype(bf16), pad], axis=-1)
    gscale = jnp.concatenate(
        [jnp.ones((F, 1), bf16),
         jnp.full((F, 1), -1.4426950408889634, bf16)], axis=0)
    Wm = Wm * gscale[None]
    c0bias = blk["c0b"] + blk["c0w"] @ init_b
    c0 = jnp.concatenate(
        [blk["c0w"], c0bias, jnp.zeros((F, 7), jnp.float32)], axis=-1)
    rwcat = blk["rw"].astype(bf16).transpose(1, 0, 2).reshape(F, L * F)
    rbsum = blk["rb"].sum(0)
    return c0, Wm, rwcat, rbsum


def _forward(x, init_w, init_b, blocks, out_size):
    B, Cin, T = x.shape
    N = B * Cin
    K, F, _ = init_w.shape
    nst = len(blocks)
    L = blocks[0]["fw"].shape[0] // K
    bf16 = jnp.bfloat16
    rows = 2
    assert Cin == rows and N % rows == 0

    prepped = [_prep_block(b, init_b, L, F) for b in blocks]
    c0w = jnp.stack([p[0] for p in prepped]).astype(bf16)
    W = jnp.stack([p[1] for p in prepped])
    rwcat = jnp.stack([p[2] for p in prepped])
    rbsum = jnp.stack([p[3] for p in prepped])
    P = jnp.asarray(_pool_matrix(T, out_size)).astype(bf16)
    xr = x.reshape(N, 1, T).astype(jnp.float32)

    kern = functools.partial(_enc_kernel, num_layers=L, rows=rows)
    out = pl.pallas_call(
        kern,
        grid=(nst, N // rows),
        in_specs=[
            pl.BlockSpec((rows, 1, T), lambda s, q: (q, 0, 0)),
            pl.BlockSpec((K, F, 1), lambda s, q: (0, 0, 0)),
            pl.BlockSpec((1, F, F + 8), lambda s, q: (s, 0, 0)),
            pl.BlockSpec((1, L, 2 * F, 3 * F + 8), lambda s, q: (s, 0, 0, 0)),
            pl.BlockSpec((1, F, L * F), lambda s, q: (s, 0, 0)),
            pl.BlockSpec((1, F, 1), lambda s, q: (s, 0, 0)),
            pl.BlockSpec((T, out_size), lambda s, q: (0, 0)),
        ],
        out_specs=pl.BlockSpec((1, 1, rows * F, out_size),
                               lambda s, q: (q, s, 0, 0)),
        out_shape=jax.ShapeDtypeStruct((B, nst, Cin * F, out_size),
                                       jnp.float32),
        compiler_params=pltpu.CompilerParams(
            dimension_semantics=("parallel", "parallel")),
    )(xr, init_w, c0w, W, rwcat, rbsum, P)
    return out


def kernel(x, init_w, init_b,
           b0_c0w, b0_c0b, b0_fw, b0_fb, b0_gw, b0_gb, b0_rw, b0_rb,
           b1_c0w, b1_c0b, b1_fw, b1_fb, b1_gw, b1_gb, b1_rw, b1_rb,
           b2_c0w, b2_c0b, b2_fw, b2_fb, b2_gw, b2_gb, b2_rw, b2_rb,
           b3_c0w, b3_c0b, b3_fw, b3_fb, b3_gw, b3_gb, b3_rw, b3_rb):
    blocks = [
        {"c0w": b0_c0w, "c0b": b0_c0b, "fw": b0_fw, "fb": b0_fb,
         "gw": b0_gw, "gb": b0_gb, "rw": b0_rw, "rb": b0_rb},
        {"c0w": b1_c0w, "c0b": b1_c0b, "fw": b1_fw, "fb": b1_fb,
         "gw": b1_gw, "gb": b1_gb, "rw": b1_rw, "rb": b1_rb},
        {"c0w": b2_c0w, "c0b": b2_c0b, "fw": b2_fw, "fb": b2_fb,
         "gw": b2_gw, "gb": b2_gb, "rw": b2_rw, "rb": b2_rb},
        {"c0w": b3_c0w, "c0b": b3_c0b, "fw": b3_fw, "fb": b3_fb,
         "gw": b3_gw, "gb": b3_gb, "rw": b3_rw, "rb": b3_rb},
    ]
    return _forward(x, init_w, init_b, blocks, 320)
